# baseline (speedup 1.0000x reference)
import functools

import jax
import jax.numpy as jnp
from jax import lax
from jax.experimental import pallas as pl
from jax.experimental.pallas import tpu as pltpu
from jax.experimental.pallas import tpu_sc as plsc

_NUM_WORKERS = 32


def _sc_gather(table, idx):
    b = idx.shape[0]
    d = table.shape[1]
    b_per_w = b // _NUM_WORKERS
    mesh = plsc.VectorSubcoreMesh(core_axis_name="c", subcore_axis_name="s")

    @functools.partial(
        pl.kernel,
        out_type=jax.ShapeDtypeStruct((b, d), jnp.float32),
        mesh=mesh,
        compiler_params=pltpu.CompilerParams(use_tc_tiling_on_sc=False),
        scratch_types=[
            pltpu.VMEM((b_per_w,), jnp.int32),
            pltpu.VMEM((b_per_w, d), jnp.float32),
            pltpu.SemaphoreType.DMA,
        ],
    )
    def k(table_hbm, idx_hbm, out_hbm, idx_v, rows_v, sem):
        wid = lax.axis_index("s") * 2 + lax.axis_index("c")
        base = wid * b_per_w
        pltpu.sync_copy(idx_hbm.at[pl.ds(base, b_per_w)], idx_v)
        pltpu.async_copy(table_hbm.at[idx_v], rows_v, sem).wait()
        pltpu.sync_copy(rows_v, out_hbm.at[pl.ds(base, b_per_w)])

    return k(table, idx)


def _tc_body(emb_ref, dt_ref, w_ref, b_ref, out_ref):
    t = jnp.cos(dt_ref[:] * w_ref[:] + b_ref[:])
    out_ref[:] = jnp.concatenate([emb_ref[:], t], axis=-1)


def _tc_cos_concat(emb, dt, w, b):
    bsz, d = emb.shape
    dim_t = w.shape[0]
    blk = 512
    grid = (bsz // blk,)
    return pl.pallas_call(
        _tc_body,
        grid=grid,
        in_specs=[
            pl.BlockSpec((blk, d), lambda i: (i, 0)),
            pl.BlockSpec((blk, 1), lambda i: (i, 0)),
            pl.BlockSpec((1, dim_t), lambda i: (0, 0)),
            pl.BlockSpec((1, dim_t), lambda i: (0, 0)),
        ],
        out_specs=pl.BlockSpec((blk, d + dim_t), lambda i: (i, 0)),
        out_shape=jax.ShapeDtypeStruct((bsz, d + dim_t), jnp.float32),
    )(emb, dt, w.reshape(1, dim_t), b.reshape(1, dim_t))


def kernel(entities, dt, table, w, b):
    idx = entities.astype(jnp.int32)
    emb = _sc_gather(table, idx)
    return _tc_cos_concat(emb, dt.astype(jnp.float32), w, b)


# speedup vs baseline: 2.4015x; 2.4015x over previous
import functools

import jax
import jax.numpy as jnp
from jax import lax
from jax.experimental import pallas as pl
from jax.experimental.pallas import tpu as pltpu
from jax.experimental.pallas import tpu_sc as plsc

_NW = 32
_L = 16
_CHUNK = 128
_CF = 2048

_TWO_PI = 6.283185307179586
_PI = 3.141592653589793
_COS_C = (
    0.9999999738948335,
    -0.49999985130227886,
    0.04166646235582207,
    -0.0013887731795384876,
    2.4769053365277362e-05,
    -2.7075450696039624e-07,
    1.7243752160329109e-09,
)


def _cos16(x):
    r = lax.rem(x, jnp.float32(_TWO_PI))
    r = r - jnp.where(r > _PI, jnp.float32(_TWO_PI), jnp.float32(0.0))
    u = r * r
    acc = jnp.full((_L,), _COS_C[6], jnp.float32)
    for c in _COS_C[5::-1]:
        acc = acc * u + jnp.float32(c)
    return acc


def _sc_embed(table_t, tail_slab, idx, dtf, w_splat, b_splat, d):
    dt_dim, n = table_t.shape
    dim_t = w_splat.shape[0]
    bsz = idx.shape[0]
    n_full = n // 128
    n_slabs = (n + 127) // 128
    per_w, extra = divmod(n_slabs, _NW)
    max_slabs = per_w + (1 if extra else 0)
    n_pairs = (max_slabs + 1) // 2
    mesh = plsc.VectorSubcoreMesh(core_axis_name="c", subcore_axis_name="s")

    @functools.partial(
        pl.kernel,
        out_type=jax.ShapeDtypeStruct((bsz + _CHUNK, 128), jnp.float32),
        mesh=mesh,
        compiler_params=pltpu.CompilerParams(
            use_tc_tiling_on_sc=True, needs_layout_passes=False
        ),
        scratch_types=[
            pltpu.VMEM((_CF,), jnp.int32),
            pltpu.VMEM((bsz + _CHUNK,), jnp.float32),
            pltpu.VMEM((bsz + 2 * _L,), jnp.int32),
            pltpu.VMEM((bsz + 2 * _L,), jnp.int32),
            pltpu.VMEM((dt_dim, 128), jnp.float32),
            pltpu.VMEM((dt_dim, 128), jnp.float32),
            pltpu.VMEM((bsz + _L,), jnp.int32),
            pltpu.VMEM((bsz + _L,), jnp.int32),
            pltpu.VMEM((_CHUNK, 128), jnp.float32),
            pltpu.VMEM((_CHUNK,), jnp.int32),
            pltpu.VMEM((dim_t, _L), jnp.float32),
            pltpu.VMEM((dim_t, _L), jnp.float32),
            pltpu.SemaphoreType.DMA,
            pltpu.SemaphoreType.DMA,
        ],
    )
    def k(tt_hbm, tail_hbm, idx_hbm, dt_hbm, ws_hbm, bs_hbm, out_hbm,
          idc_v, dt_all, my_ids, my_pos, slab_a, slab_b, pend_l, pend_p,
          chunk_v, cpos_v, ws_v, bs_v, sem_a, sem_b):
        w = lax.axis_index("s") * 2 + lax.axis_index("c")
        iota = jnp.arange(_L, dtype=jnp.int32)

        pltpu.sync_copy(dt_hbm.at[pl.ds(0, bsz)], dt_all.at[pl.ds(0, bsz)])
        pltpu.sync_copy(ws_hbm, ws_v)
        pltpu.sync_copy(bs_hbm, bs_v)

        lo = w * per_w + jnp.minimum(w, extra)
        hi = lo + per_w + jnp.where(w < extra, 1, 0)
        rlo = lo * 128
        rhi = hi * 128

        def reset_cpos():
            def init(q, carry):
                cpos_v[pl.ds(q * _L, _L)] = bsz + q * _L + iota
                return carry

            lax.fori_loop(0, _CHUNK // _L, init, 0)

        reset_cpos()

        def coarse_chunk(cc, off):
            pltpu.sync_copy(idx_hbm.at[pl.ds(cc * _CF, _CF)], idc_v)

            def vstep(g, off2):
                for u in range(2):
                    ids16 = idc_v[pl.ds((2 * g + u) * _L, _L)]
                    m = (ids16 >= rlo) & (ids16 < rhi)
                    cnt = plsc.all_reduce_population_count(m)[0]
                    plsc.store_compressed(
                        my_ids.at[pl.ds(off2, _L)], ids16, mask=m
                    )
                    plsc.store_compressed(
                        my_pos.at[pl.ds(off2, _L)],
                        cc * _CF + (2 * g + u) * _L + iota,
                        mask=m,
                    )
                    off2 = off2 + cnt
                return off2

            return lax.fori_loop(0, _CF // (2 * _L), vstep, off)

        k_w = lax.fori_loop(0, bsz // _CF, coarse_chunk, jnp.int32(0))
        sent = jnp.full((_L,), -1, jnp.int32)
        my_ids[pl.ds(k_w, _L)] = sent
        my_ids[pl.ds(k_w + _L, _L)] = sent
        n_sweep = (k_w + 2 * _L - 1) // (2 * _L)

        def flush():
            def cos_group(g8, carry):
                cpos16 = cpos_v[pl.ds(g8 * _L, _L)]
                dt16 = plsc.load_gather(dt_all, [cpos16])
                rvec = g8 * _L + iota
                for j in range(dim_t):
                    t = _cos16(dt16 * ws_v[j] + bs_v[j])
                    plsc.store_scatter(
                        chunk_v, [rvec, jnp.full((_L,), d + j, jnp.int32)], t
                    )
                return carry

            lax.fori_loop(0, _CHUNK // _L, cos_group, 0)
            pltpu.sync_copy(chunk_v, out_hbm.at[cpos_v])

        def issue(jj, buf, sem):
            j = lo + jj

            @pl.when(j < n_full)
            def _full():
                col0 = pl.multiple_of(j * 128, 128)
                pltpu.async_copy(tt_hbm.at[:, pl.ds(col0, 128)], buf, sem)

            @pl.when(j >= n_full)
            def _tail():
                pltpu.async_copy(tail_hbm, buf, sem)

        def drain(buf, sem):
            pltpu.make_async_copy(
                tt_hbm.at[:, pl.ds(0, 128)], buf, sem
            ).wait()

        def process(j, slab_v, c):
            base = j * 128

            def sweep(gg, poff):
                for u in range(2):
                    o = (2 * gg + u) * _L
                    ids16 = my_ids[pl.ds(o, _L)]
                    m = (ids16 >= base) & (ids16 < base + 128)
                    cnt = plsc.all_reduce_population_count(m)[0]
                    plsc.store_compressed(
                        pend_l.at[pl.ds(poff, _L)], ids16 - base, mask=m
                    )
                    pos16 = my_pos[pl.ds(o, _L)]
                    plsc.store_compressed(
                        pend_p.at[pl.ds(poff, _L)], pos16, mask=m
                    )
                    poff = poff + cnt
                return poff

            kp = lax.fori_loop(0, n_sweep, sweep, jnp.int32(0))

            def group(q, c3):
                cnt16 = jnp.minimum(kp - q * _L, _L)

                @pl.when(c3 + cnt16 > _CHUNK)
                def _spill():
                    flush()
                    reset_cpos()

                c3 = jnp.where(c3 + cnt16 > _CHUNK, 0, c3)
                lanes16 = pend_l[pl.ds(q * _L, _L)]
                pos16 = pend_p[pl.ds(q * _L, _L)]
                valid = iota < cnt16
                rows = c3 + iota
                plsc.store_scatter(cpos_v, [rows], pos16, mask=valid)
                for col in range(d):
                    vals = plsc.load_gather(
                        slab_v,
                        [jnp.full((_L,), col, jnp.int32), lanes16],
                        mask=valid,
                    )
                    plsc.store_scatter(
                        chunk_v,
                        [rows, jnp.full((_L,), col, jnp.int32)],
                        vals,
                        mask=valid,
                    )
                return c3 + cnt16

            return lax.fori_loop(0, (kp + _L - 1) // _L, group, c)

        issue(0, slab_a, sem_a)

        def pair(i2, c):
            for p in range(2):
                jj = i2 * 2 + p
                cur, csem = (slab_a, sem_a) if p == 0 else (slab_b, sem_b)
                nxt, nsem = (slab_b, sem_b) if p == 0 else (slab_a, sem_a)
                j = lo + jj

                @pl.when(j < hi)
                def _run():
                    drain(cur, csem)

                    @pl.when(j + 1 < hi)
                    def _pre():
                        issue(jj + 1, nxt, nsem)

                c = lax.cond(
                    j < hi,
                    functools.partial(process, j, cur),
                    lambda c_: c_,
                    c,
                )
            return c

        lax.fori_loop(0, n_pairs, pair, jnp.int32(0))
        flush()

    return k(table_t, tail_slab, idx, dtf, w_splat, b_splat)


def kernel(entities, dt, table, w, b):
    idx = entities.astype(jnp.int32)
    dtf = dt.astype(jnp.float32).reshape(-1)
    n, d = table.shape
    dim_t = w.shape[0]
    n_full = n // 128
    tail_slab = jnp.zeros((d, 128), jnp.float32)
    tail_slab = lax.dynamic_update_slice(
        tail_slab, table.T[:, n_full * 128:], (0, 0)
    )
    w_splat = jnp.broadcast_to(
        w.astype(jnp.float32).reshape(dim_t, 1), (dim_t, _L)
    )
    b_splat = jnp.broadcast_to(
        b.astype(jnp.float32).reshape(dim_t, 1), (dim_t, _L)
    )
    padded = _sc_embed(table.T, tail_slab, idx, dtf, w_splat, b_splat, d)
    return padded[: idx.shape[0], : d + dim_t]
